# Initial kernel scaffold; baseline (speedup 1.0000x reference)
import jax
import jax.numpy as jnp
from jax.experimental import pallas as pl


def kernel(hidden_states, attention_mask, position_ids, Wq, Wk, Wv, Wo):
    raise NotImplementedError("write your pallas kernel here")


# speedup vs baseline: 2.7081x; 2.7081x over previous
import functools
import math

import jax
import jax.numpy as jnp
from jax.experimental import pallas as pl
from jax.experimental.pallas import tpu as pltpu


def _qkv_rope_kernel(x_ref, w_ref, cos_ref, sin_ref, o_ref, *, n_rope_blocks, hd):
    z = jax.lax.dot_general(
        x_ref[...], w_ref[...], (((1,), (0,)), ((), ())),
        preferred_element_type=jnp.float32)
    bs, bc = z.shape
    nh = bc // hd
    z3 = z.reshape(bs, nh, hd)
    half = hd // 2
    zr = jnp.concatenate([-z3[..., half:], z3[..., :half]], axis=-1)
    cos = cos_ref[...][:, None, :]
    sin = sin_ref[...][:, None, :]
    roped = z3 * cos + zr * sin
    use_rope = pl.program_id(0) < n_rope_blocks
    o_ref[...] = jnp.where(use_rope, roped, z3).reshape(bs, bc)


def _matmul_kernel(x_ref, w_ref, o_ref):
    o_ref[...] = jax.lax.dot_general(
        x_ref[...], w_ref[...], (((1,), (0,)), ((), ())),
        preferred_element_type=jnp.float32)


def _attn_kernel(q_ref, k_ref, v_ref, o_ref, a_scr, p_scr, acc_ref,
                 *, bsq, s, hb, rb, scale):
    j = pl.program_id(1)

    @pl.when(j == 0)
    def _():
        acc_ref[...] = jnp.zeros_like(acc_ref)

    a = jax.lax.dot_general(
        q_ref[...], k_ref[...], (((1,), (1,)), ((), ())),
        preferred_element_type=jnp.float32) * scale
    a = jnp.maximum(a, jnp.finfo(jnp.float32).min)
    a_scr[...] = a
    mrow = jnp.max(a, axis=1, keepdims=True)
    e = jnp.exp(a - mrow)
    p_scr[...] = e / jnp.sum(e, axis=1, keepdims=True)

    col = jax.lax.broadcasted_iota(jnp.int32, (1, s), 1)

    def body(r, carry):
        i = j * bsq + r
        a_i = a_scr[pl.ds(r, 1), :]
        p_i = p_scr[pl.ds(r, 1), :]
        acc = acc_ref[...]
        support = acc > 0.0
        m = jnp.min(jnp.where(support, acc, jnp.inf))
        jstar = jnp.max(jnp.where(acc == m, col, -1))
        mbi = (support & (col != jstar)) | (col == i)
        is_init = i < hb
        upd = jnp.where(is_init, (col < hb).astype(jnp.float32),
                        mbi.astype(jnp.float32))
        acc_ref[...] = (acc + p_i) * upd
        causal = col <= i
        out_init = jnp.where(causal, a_i, 0.0)
        out_scan = jnp.where((mbi | (col >= i - rb)) & causal, a_i, 0.0)
        a_scr[pl.ds(r, 1), :] = jnp.where(is_init, out_init, out_scan)
        return carry

    jax.lax.fori_loop(0, bsq, body, 0)

    am = a_scr[...]
    mr = jnp.max(am, axis=1, keepdims=True)
    w = jnp.exp(am - mr)
    w = w / jnp.sum(w, axis=1, keepdims=True)
    o_ref[...] = jax.lax.dot_general(
        w, v_ref[...], (((1,), (0,)), ((), ())),
        preferred_element_type=jnp.float32)


def _forward(hidden_states, attention_mask, position_ids, Wq, Wk, Wv, Wo, nheads):
    b, s, d = hidden_states.shape
    hd = d // nheads
    hb = int(0.1 * s)
    rb = int(0.1 * s)

    x = hidden_states[0]
    wt = jnp.concatenate([Wq.T, Wk.T, Wv.T], axis=1)

    inv_freq = 1.0 / (10000.0 ** (jnp.arange(0, hd, 2, dtype=jnp.float32) / hd))
    t = jnp.arange(s, dtype=jnp.float32)
    freqs = jnp.outer(t, inv_freq)
    emb = jnp.concatenate([freqs, freqs], axis=-1)
    pos = position_ids[0]
    cos = jnp.take(jnp.cos(emb), pos, axis=0)
    sin = jnp.take(jnp.sin(emb), pos, axis=0)

    bs = min(256, s)
    bc = min(1024, d)
    ncol = (3 * d) // bc
    nrow = s // bs
    n_rope_blocks = (2 * d) // bc

    qkv = pl.pallas_call(
        functools.partial(_qkv_rope_kernel, n_rope_blocks=n_rope_blocks, hd=hd),
        grid=(ncol, nrow),
        in_specs=[
            pl.BlockSpec((bs, d), lambda c, r: (r, 0)),
            pl.BlockSpec((d, bc), lambda c, r: (0, c)),
            pl.BlockSpec((bs, hd), lambda c, r: (r, 0)),
            pl.BlockSpec((bs, hd), lambda c, r: (r, 0)),
        ],
        out_specs=pl.BlockSpec((bs, bc), lambda c, r: (r, c)),
        out_shape=jax.ShapeDtypeStruct((s, 3 * d), jnp.float32),
        compiler_params=pltpu.CompilerParams(
            dimension_semantics=("arbitrary", "arbitrary")),
    )(x, wt, cos, sin)

    bsq = min(256, s)
    nj = s // bsq
    scale = 1.0 / math.sqrt(hd)

    o1 = pl.pallas_call(
        functools.partial(_attn_kernel, bsq=bsq, s=s, hb=hb, rb=rb, scale=scale),
        grid=(nheads, nj),
        in_specs=[
            pl.BlockSpec((bsq, hd), lambda h, j: (j, h)),
            pl.BlockSpec((s, hd), lambda h, j, n=nheads: (0, n + h)),
            pl.BlockSpec((s, hd), lambda h, j, n=nheads: (0, 2 * n + h)),
        ],
        out_specs=pl.BlockSpec((bsq, hd), lambda h, j: (j, h)),
        out_shape=jax.ShapeDtypeStruct((s, d), jnp.float32),
        scratch_shapes=[
            pltpu.VMEM((bsq, s), jnp.float32),
            pltpu.VMEM((bsq, s), jnp.float32),
            pltpu.VMEM((1, s), jnp.float32),
        ],
        compiler_params=pltpu.CompilerParams(
            dimension_semantics=("arbitrary", "arbitrary")),
    )(qkv, qkv, qkv)

    bco = min(1024, d)
    o2 = pl.pallas_call(
        _matmul_kernel,
        grid=(d // bco, nrow),
        in_specs=[
            pl.BlockSpec((bs, d), lambda c, r: (r, 0)),
            pl.BlockSpec((d, bco), lambda c, r: (0, c)),
        ],
        out_specs=pl.BlockSpec((bs, bco), lambda c, r: (r, c)),
        out_shape=jax.ShapeDtypeStruct((s, d), jnp.float32),
        compiler_params=pltpu.CompilerParams(
            dimension_semantics=("arbitrary", "arbitrary")),
    )(o1, Wo.T)

    return o2.reshape(b, s, d)


def kernel(hidden_states, attention_mask, position_ids, Wq, Wk, Wv, Wo):
    return _forward(hidden_states, attention_mask, position_ids,
                    Wq, Wk, Wv, Wo, 16)


# speedup vs baseline: 3.1526x; 1.1641x over previous
import functools
import math

import jax
import jax.numpy as jnp
from jax.experimental import pallas as pl
from jax.experimental.pallas import tpu as pltpu


def _qkv_rope_kernel(x_ref, w_ref, cos_ref, sin_ref, o_ref, *, n_rope_blocks, hd):
    z = jax.lax.dot_general(
        x_ref[...], w_ref[...], (((1,), (0,)), ((), ())),
        preferred_element_type=jnp.float32)
    bs, bc = z.shape
    nh = bc // hd
    z3 = z.reshape(bs, nh, hd)
    half = hd // 2
    zr = jnp.concatenate([-z3[..., half:], z3[..., :half]], axis=-1)
    cos = cos_ref[...][:, None, :]
    sin = sin_ref[...][:, None, :]
    roped = z3 * cos + zr * sin
    use_rope = pl.program_id(0) < n_rope_blocks
    o_ref[...] = jnp.where(use_rope, roped, z3).reshape(bs, bc)


def _matmul_kernel(x_ref, w_ref, o_ref):
    o_ref[...] = jax.lax.dot_general(
        x_ref[...], w_ref[...], (((1,), (0,)), ((), ())),
        preferred_element_type=jnp.float32)


def _attn_kernel(q_ref, k_ref, v_ref, o_ref, a_scr, p_scr, acc_ref,
                 *, bsq, s, hb, rb, scale):
    j = pl.program_id(1)

    @pl.when(j == 0)
    def _():
        acc_ref[...] = jnp.zeros_like(acc_ref)

    a = jax.lax.dot_general(
        q_ref[...], k_ref[...], (((1,), (1,)), ((), ())),
        preferred_element_type=jnp.float32) * scale
    a = jnp.maximum(a, jnp.finfo(jnp.float32).min)
    a_scr[...] = a
    mrow = jnp.max(a, axis=1, keepdims=True)
    e = jnp.exp(a - mrow)
    p = e / jnp.sum(e, axis=1, keepdims=True)
    p_scr[...] = p

    col = jax.lax.broadcasted_iota(jnp.int32, (1, s), 1)

    @pl.when(j * bsq < hb)
    def _():
        row = jax.lax.broadcasted_iota(jnp.int32, (bsq, s), 0) + j * bsq
        colb = jax.lax.broadcasted_iota(jnp.int32, (bsq, s), 1)
        init_rows = row < hb
        a_scr[...] = jnp.where((colb <= row) | jnp.logical_not(init_rows),
                               a, 0.0)
        contrib = jnp.sum(jnp.where(init_rows, p, 0.0), axis=0, keepdims=True)
        acc_ref[...] = acc_ref[...] + contrib * (col < hb).astype(jnp.float32)

    def body(r, carry):
        i = j * bsq + r
        a_i = a_scr[pl.ds(r, 1), :]
        p_i = p_scr[pl.ds(r, 1), :]
        acc = acc_ref[...]
        support = acc > 0.0
        m = jnp.min(jnp.where(support, acc, jnp.inf))
        jstar = jnp.max(jnp.where(acc == m, col, -1))
        mbi = (support & (col != jstar)) | (col == i)
        acc_ref[...] = (acc + p_i) * mbi.astype(jnp.float32)
        keep = (mbi | (col >= i - rb)) & (col <= i)
        a_scr[pl.ds(r, 1), :] = jnp.where(keep, a_i, 0.0)
        return carry

    r0 = jnp.maximum(hb - j * bsq, 0)
    jax.lax.fori_loop(r0, bsq, body, 0)

    am = a_scr[...]
    mr = jnp.max(am, axis=1, keepdims=True)
    w = jnp.exp(am - mr)
    w = w / jnp.sum(w, axis=1, keepdims=True)
    o_ref[...] = jax.lax.dot_general(
        w, v_ref[...], (((1,), (0,)), ((), ())),
        preferred_element_type=jnp.float32)


def _forward(hidden_states, attention_mask, position_ids, Wq, Wk, Wv, Wo, nheads):
    b, s, d = hidden_states.shape
    hd = d // nheads
    hb = int(0.1 * s)
    rb = int(0.1 * s)

    x = hidden_states[0]
    wt = jnp.concatenate([Wq.T, Wk.T, Wv.T], axis=1)

    inv_freq = 1.0 / (10000.0 ** (jnp.arange(0, hd, 2, dtype=jnp.float32) / hd))
    t = jnp.arange(s, dtype=jnp.float32)
    freqs = jnp.outer(t, inv_freq)
    emb = jnp.concatenate([freqs, freqs], axis=-1)
    pos = position_ids[0]
    cos = jnp.take(jnp.cos(emb), pos, axis=0)
    sin = jnp.take(jnp.sin(emb), pos, axis=0)

    bs = min(256, s)
    bc = min(1024, d)
    ncol = (3 * d) // bc
    nrow = s // bs
    n_rope_blocks = (2 * d) // bc

    qkv = pl.pallas_call(
        functools.partial(_qkv_rope_kernel, n_rope_blocks=n_rope_blocks, hd=hd),
        grid=(ncol, nrow),
        in_specs=[
            pl.BlockSpec((bs, d), lambda c, r: (r, 0)),
            pl.BlockSpec((d, bc), lambda c, r: (0, c)),
            pl.BlockSpec((bs, hd), lambda c, r: (r, 0)),
            pl.BlockSpec((bs, hd), lambda c, r: (r, 0)),
        ],
        out_specs=pl.BlockSpec((bs, bc), lambda c, r: (r, c)),
        out_shape=jax.ShapeDtypeStruct((s, 3 * d), jnp.float32),
        compiler_params=pltpu.CompilerParams(
            dimension_semantics=("parallel", "parallel")),
    )(x, wt, cos, sin)

    bsq = min(256, s)
    nj = s // bsq
    scale = 1.0 / math.sqrt(hd)

    o1 = pl.pallas_call(
        functools.partial(_attn_kernel, bsq=bsq, s=s, hb=hb, rb=rb, scale=scale),
        grid=(nheads, nj),
        in_specs=[
            pl.BlockSpec((bsq, hd), lambda h, j: (j, h)),
            pl.BlockSpec((s, hd), lambda h, j, n=nheads: (0, n + h)),
            pl.BlockSpec((s, hd), lambda h, j, n=nheads: (0, 2 * n + h)),
        ],
        out_specs=pl.BlockSpec((bsq, hd), lambda h, j: (j, h)),
        out_shape=jax.ShapeDtypeStruct((s, d), jnp.float32),
        scratch_shapes=[
            pltpu.VMEM((bsq, s), jnp.float32),
            pltpu.VMEM((bsq, s), jnp.float32),
            pltpu.VMEM((1, s), jnp.float32),
        ],
        compiler_params=pltpu.CompilerParams(
            dimension_semantics=("parallel", "arbitrary")),
    )(qkv, qkv, qkv)

    bco = min(1024, d)
    o2 = pl.pallas_call(
        _matmul_kernel,
        grid=(d // bco, nrow),
        in_specs=[
            pl.BlockSpec((bs, d), lambda c, r: (r, 0)),
            pl.BlockSpec((d, bco), lambda c, r: (0, c)),
        ],
        out_specs=pl.BlockSpec((bs, bco), lambda c, r: (r, c)),
        out_shape=jax.ShapeDtypeStruct((s, d), jnp.float32),
        compiler_params=pltpu.CompilerParams(
            dimension_semantics=("parallel", "parallel")),
    )(o1, Wo.T)

    return o2.reshape(b, s, d)


def kernel(hidden_states, attention_mask, position_ids, Wq, Wk, Wv, Wo):
    return _forward(hidden_states, attention_mask, position_ids,
                    Wq, Wk, Wv, Wo, 16)
